# baseline (speedup 1.0000x reference)
import functools

import jax
import jax.numpy as jnp
from jax import lax
from jax.experimental import pallas as pl
from jax.experimental.pallas import tpu as pltpu
from jax.experimental.pallas import tpu_sc as plsc

L = 16
E = 16
NS = 16
CHUNK = 128


def _sc_forward(n_rows_table):
    B = 16384
    R = B // NS
    C = R // CHUNK
    RB = B // CHUNK

    mesh = plsc.VectorSubcoreMesh(core_axis_name="c", subcore_axis_name="s",
                                  num_cores=1)

    @functools.partial(
        pl.kernel,
        out_type=jax.ShapeDtypeStruct((RB, CHUNK), jnp.float32),
        mesh=mesh,
        compiler_params=pltpu.CompilerParams(use_tc_tiling_on_sc=False),
        scratch_types=[
            pltpu.VMEM((C, CHUNK), jnp.int32),
            pltpu.VMEM((C, CHUNK), jnp.int32),
            pltpu.VMEM((C, CHUNK, E), jnp.float32),
            pltpu.VMEM((C, CHUNK, E), jnp.float32),
            pltpu.VMEM((C, CHUNK), jnp.float32),
            pltpu.VMEM((C, CHUNK), jnp.float32),
            pltpu.VMEM((C, CHUNK), jnp.float32),
            pltpu.VMEM((L,), jnp.float32),
            pltpu.VMEM((NS, L), jnp.float32),
            pltpu.VMEM_SHARED((NS, L), jnp.float32),
            pltpu.SemaphoreType.DMA,
        ],
    )
    def body(uid_h, iid_h, ue_h, ub_h, ie_h, ib_h, out_h,
             uid_v, iid_v, ur_v, ir_v, ubv, ibv, outv, accv, allp, shr, sem):
        sid = lax.axis_index("s")
        row0 = sid * C

        pltpu.sync_copy(uid_h.at[pl.ds(row0, C)], uid_v)
        pltpu.sync_copy(iid_h.at[pl.ds(row0, C)], iid_v)

        copies = []
        for c in range(C):
            copies.append(pltpu.async_copy(
                ue_h.at[uid_v.at[c]], ur_v.at[c], sem))
            copies.append(pltpu.async_copy(
                ie_h.at[iid_v.at[c]], ir_v.at[c], sem))
        for c in range(C):
            copies.append(pltpu.async_copy(
                ub_h.at[uid_v.at[c]], ubv.at[c], sem))
            copies.append(pltpu.async_copy(
                ib_h.at[iid_v.at[c]], ibv.at[c], sem))
        for cp in copies:
            cp.wait()

        def dot_chunk(c):
            def dot_row(r, acc):
                return acc + ur_v[c, r] * ir_v[c, r]
            return lax.fori_loop(0, CHUNK, dot_row,
                                 jnp.zeros((L,), jnp.float32))

        acc = dot_chunk(0)
        for c in range(1, C):
            acc = acc + dot_chunk(c)
        accv[...] = acc

        pltpu.sync_copy(accv, shr.at[sid])
        plsc.subcore_barrier()
        pltpu.sync_copy(shr, allp)
        tot = allp[0]
        for j in range(1, NS):
            tot = tot + allp[j]
        lanes = lax.iota(jnp.int32, L)
        for shift in (1, 2, 4, 8):
            tot = tot + tot.at[(lanes + shift) % L].get(
                mode="promise_in_bounds")
        s = tot

        def out_chunk(c, k):
            x = s + ubv[c, pl.ds(k * L, L)] + ibv[c, pl.ds(k * L, L)]
            outv[c, pl.ds(k * L, L)] = 1.0 / (1.0 + jnp.exp(-x))

        for c in range(C):
            lax.fori_loop(0, CHUNK // L,
                          lambda k, _, c=c: (out_chunk(c, k), 0)[1], 0)

        pltpu.sync_copy(outv, out_h.at[pl.ds(row0, C)])

    return body


def kernel(inputs, user_embedding, user_bias, item_embedding, item_bias):
    B = inputs.shape[0]
    ii = inputs.astype(jnp.int32)
    uid = ii[:, 0].reshape(B // CHUNK, CHUNK)
    iid = ii[:, 1].reshape(B // CHUNK, CHUNK)
    ub = user_bias.reshape(-1)
    ib = item_bias.reshape(-1)
    fwd = _sc_forward(user_embedding.shape[0])
    out = fwd(uid, iid, user_embedding, ub, item_embedding, ib)
    return out.reshape(B, 1)
